# Initial kernel scaffold; baseline (speedup 1.0000x reference)
import functools

import jax
import jax.numpy as jnp
from jax import lax
from jax.experimental import pallas as pl
from jax.experimental.pallas import tpu as pltpu
from jax.experimental.pallas import tpu_sc as plsc

_CH = 128
_NSUB = 16
_NCORE = 2
_BLK = 1024


def _prep_body(x_ref, w_ref, as_ref, ad_ref, h_ref, tas_ref, tad_ref, c_ref,
               ms_ref):
    i = pl.program_id(0)
    g = pl.num_programs(0)
    h = jnp.dot(x_ref[...], w_ref[...], preferred_element_type=jnp.float32)
    h_ref[...] = h
    a_s = jnp.dot(h, as_ref[...], preferred_element_type=jnp.float32)
    a_d = jnp.dot(h, ad_ref[...], preferred_element_type=jnp.float32)
    tas_ref[...] = a_s
    tad_ref[...] = a_d
    prev_s = jnp.where(i == 0, -jnp.inf, ms_ref[0])
    prev_d = jnp.where(i == 0, -jnp.inf, ms_ref[1])
    ms_ref[0] = jnp.maximum(prev_s, jnp.max(a_s))
    ms_ref[1] = jnp.maximum(prev_d, jnp.max(a_d))

    @pl.when(i == g - 1)
    def _():
        c_ref[...] = jnp.full((1, 128), ms_ref[0] + ms_ref[1], jnp.float32)


def _prep(xp, W1, A1s, A1d):
    NP, F = xp.shape
    D = W1.shape[1]
    grid = (NP // _BLK,)
    return pl.pallas_call(
        _prep_body,
        grid=grid,
        in_specs=[
            pl.BlockSpec((_BLK, F), lambda i: (i, 0)),
            pl.BlockSpec((F, D), lambda i: (0, 0)),
            pl.BlockSpec((D, 16), lambda i: (0, 0)),
            pl.BlockSpec((D, 16), lambda i: (0, 0)),
        ],
        out_specs=[
            pl.BlockSpec((_BLK, D), lambda i: (i, 0)),
            pl.BlockSpec((_BLK, 16), lambda i: (i, 0)),
            pl.BlockSpec((_BLK, 16), lambda i: (i, 0)),
            pl.BlockSpec((1, 128), lambda i: (0, 0)),
        ],
        out_shape=[
            jax.ShapeDtypeStruct((NP, D), jnp.float32),
            jax.ShapeDtypeStruct((NP, 16), jnp.float32),
            jax.ShapeDtypeStruct((NP, 16), jnp.float32),
            jax.ShapeDtypeStruct((1, 128), jnp.float32),
        ],
        scratch_shapes=[pltpu.SMEM((2,), jnp.float32)],
    )(xp, W1, A1s, A1d)


def _make_edge_fn(NP, EP, heads, dim):
    nw = _NCORE * _NSUB
    chunks_pw = EP // (nw * _CH)
    rows_ps = NP // _NSUB
    zch = rows_ps // _CH
    nv = dim // 16
    lanes = dim // heads // 16
    mesh = plsc.VectorSubcoreMesh(core_axis_name="c", subcore_axis_name="s",
                                  num_cores=_NCORE, num_subcores=_NSUB)

    def body(src_ref, dst_ref, tab_ref, tas_ref, tad_ref, c_ref,
             nump_ref, denp_ref,
             acc_msg, acc_den, vsrc, vdst, rows, asr, adr, eden, cbuf,
             sem_h, sem_a, sem_b):
        cid = lax.axis_index("c")
        sid = lax.axis_index("s")

        def zrow(i, _):
            for j in range(nv):
                rows[i, pl.ds(j * 16, 16)] = jnp.zeros((16,), jnp.float32)
            eden[i] = jnp.zeros((16,), jnp.float32)
            return 0

        lax.fori_loop(0, _CH, zrow, 0)

        def zacc(k, _):
            r = sid * rows_ps + k * _CH
            pltpu.sync_copy(rows, acc_msg.at[pl.ds(r, _CH)])
            pltpu.sync_copy(eden, acc_den.at[pl.ds(r, _CH)])
            return 0

        lax.fori_loop(0, zch, zacc, 0)
        pltpu.sync_copy(c_ref, cbuf)
        plsc.subcore_barrier()
        cval = cbuf[0]
        wid = cid * _NSUB + sid

        def chunk(k, _):
            row = wid * chunks_pw + k
            pltpu.sync_copy(src_ref.at[row], vsrc)
            pltpu.sync_copy(dst_ref.at[row], vdst)
            cph = pltpu.async_copy(tab_ref.at[vsrc], rows, sem_h)
            cpa = pltpu.async_copy(tas_ref.at[vsrc], asr, sem_a)
            cpb = pltpu.async_copy(tad_ref.at[vdst], adr, sem_b)
            cpa.wait()
            cpb.wait()

            def erow(i, _):
                t = asr[i] + adr[i]
                alpha = jnp.maximum(t, 0.2 * t)
                eden[i] = jnp.exp(alpha - cval)
                return 0

            lax.fori_loop(0, _CH, erow, 0)
            cph.wait()

            def mrow(i, _):
                for hh in range(heads):
                    ev = eden[i, hh]
                    for j in range(lanes):
                        off = hh * lanes * 16 + j * 16
                        rows[i, pl.ds(off, 16)] = rows[i, pl.ds(off, 16)] * ev
                return 0

            lax.fori_loop(0, _CH, mrow, 0)
            pltpu.sync_copy(eden, acc_den.at[vdst], add=True)
            pltpu.sync_copy(rows, acc_msg.at[vdst], add=True)
            return 0

        lax.fori_loop(0, chunks_pw, chunk, 0)
        plsc.subcore_barrier()

        def wout(k, _):
            r = sid * rows_ps + k * _CH
            pltpu.sync_copy(acc_msg.at[pl.ds(r, _CH)], rows)
            pltpu.sync_copy(rows, nump_ref.at[cid, pl.ds(r, _CH)])
            pltpu.sync_copy(acc_den.at[pl.ds(r, _CH)], eden)
            pltpu.sync_copy(eden, denp_ref.at[cid, pl.ds(r, _CH)])
            return 0

        lax.fori_loop(0, zch, wout, 0)

    return pl.kernel(
        body,
        out_type=[
            jax.ShapeDtypeStruct((_NCORE, NP, dim), jnp.float32),
            jax.ShapeDtypeStruct((_NCORE, NP, 16), jnp.float32),
        ],
        mesh=mesh,
        scratch_types=[
            pltpu.VMEM_SHARED((NP, dim), jnp.float32),
            pltpu.VMEM_SHARED((NP, 16), jnp.float32),
            pltpu.VMEM((_CH,), jnp.int32),
            pltpu.VMEM((_CH,), jnp.int32),
            pltpu.VMEM((_CH, dim), jnp.float32),
            pltpu.VMEM((_CH, 16), jnp.float32),
            pltpu.VMEM((_CH, 16), jnp.float32),
            pltpu.VMEM((_CH, 16), jnp.float32),
            pltpu.VMEM((16,), jnp.float32),
            pltpu.SemaphoreType.DMA,
            pltpu.SemaphoreType.DMA,
            pltpu.SemaphoreType.DMA,
        ],
    )


def _mid_body(n0_ref, n1_ref, d0_ref, d1_ref, r1_ref, b1_ref, w2_ref,
              as2_ref, ad2_ref,
              out1_ref, h2_ref, tas_ref, tad_ref, c_ref, ms_ref):
    i = pl.program_id(0)
    g = pl.num_programs(0)
    num = n0_ref[...] + n1_ref[...]
    den = d0_ref[...] + d1_ref[...]
    denf = jnp.dot(den, r1_ref[...], preferred_element_type=jnp.float32)
    o1 = num / (denf + 1e-16) + b1_ref[...]
    out1_ref[...] = o1
    gact = jnp.where(o1 > 0, o1, jnp.expm1(o1))
    h2 = jnp.dot(gact, w2_ref[...], preferred_element_type=jnp.float32)
    h2_ref[...] = h2
    a_s = jnp.dot(h2, as2_ref[...], preferred_element_type=jnp.float32)
    a_d = jnp.dot(h2, ad2_ref[...], preferred_element_type=jnp.float32)
    tas_ref[...] = a_s
    tad_ref[...] = a_d
    prev_s = jnp.where(i == 0, -jnp.inf, ms_ref[0])
    prev_d = jnp.where(i == 0, -jnp.inf, ms_ref[1])
    ms_ref[0] = jnp.maximum(prev_s, jnp.max(a_s))
    ms_ref[1] = jnp.maximum(prev_d, jnp.max(a_d))

    @pl.when(i == g - 1)
    def _():
        c_ref[...] = jnp.full((1, 128), ms_ref[0] + ms_ref[1], jnp.float32)


def _mid(n0, n1, d0, d1, R1, b1r, W2, A2s, A2d):
    NP, D = n0.shape
    NC = W2.shape[1]
    grid = (NP // _BLK,)
    return pl.pallas_call(
        _mid_body,
        grid=grid,
        in_specs=[
            pl.BlockSpec((_BLK, D), lambda i: (i, 0)),
            pl.BlockSpec((_BLK, D), lambda i: (i, 0)),
            pl.BlockSpec((_BLK, 16), lambda i: (i, 0)),
            pl.BlockSpec((_BLK, 16), lambda i: (i, 0)),
            pl.BlockSpec((16, D), lambda i: (0, 0)),
            pl.BlockSpec((1, D), lambda i: (0, 0)),
            pl.BlockSpec((D, NC), lambda i: (0, 0)),
            pl.BlockSpec((NC, 16), lambda i: (0, 0)),
            pl.BlockSpec((NC, 16), lambda i: (0, 0)),
        ],
        out_specs=[
            pl.BlockSpec((_BLK, D), lambda i: (i, 0)),
            pl.BlockSpec((_BLK, NC), lambda i: (i, 0)),
            pl.BlockSpec((_BLK, 16), lambda i: (i, 0)),
            pl.BlockSpec((_BLK, 16), lambda i: (i, 0)),
            pl.BlockSpec((1, 128), lambda i: (0, 0)),
        ],
        out_shape=[
            jax.ShapeDtypeStruct((NP, D), jnp.float32),
            jax.ShapeDtypeStruct((NP, NC), jnp.float32),
            jax.ShapeDtypeStruct((NP, 16), jnp.float32),
            jax.ShapeDtypeStruct((NP, 16), jnp.float32),
            jax.ShapeDtypeStruct((1, 128), jnp.float32),
        ],
        scratch_shapes=[pltpu.SMEM((2,), jnp.float32)],
    )(n0, n1, d0, d1, R1, b1r, W2, A2s, A2d)


def _fin_body(n0_ref, n1_ref, d0_ref, d1_ref, r2_ref, b2_ref, out_ref):
    num = n0_ref[...] + n1_ref[...]
    den = d0_ref[...] + d1_ref[...]
    denf = jnp.dot(den, r2_ref[...], preferred_element_type=jnp.float32)
    o2 = num / (denf + 1e-16) + b2_ref[...]
    m = jnp.max(o2, axis=1, keepdims=True)
    ex = jnp.exp(o2 - m)
    lse = jnp.log(jnp.sum(ex, axis=1, keepdims=True))
    out_ref[...] = o2 - m - lse


def _fin(n0, n1, d0, d1, R2, b2r):
    NP, NC = n0.shape
    grid = (NP // _BLK,)
    return pl.pallas_call(
        _fin_body,
        grid=grid,
        in_specs=[
            pl.BlockSpec((_BLK, NC), lambda i: (i, 0)),
            pl.BlockSpec((_BLK, NC), lambda i: (i, 0)),
            pl.BlockSpec((_BLK, 16), lambda i: (i, 0)),
            pl.BlockSpec((_BLK, 16), lambda i: (i, 0)),
            pl.BlockSpec((16, NC), lambda i: (0, 0)),
            pl.BlockSpec((1, NC), lambda i: (0, 0)),
        ],
        out_specs=[pl.BlockSpec((_BLK, NC), lambda i: (i, 0))],
        out_shape=[jax.ShapeDtypeStruct((NP, NC), jnp.float32)],
    )(n0, n1, d0, d1, R2, b2r)


def kernel(x, edge_index, encoder_type, W1, att_src1, att_dst1, b1,
           W2, att_src2, att_dst2, b2):
    del encoder_type
    N, F = x.shape
    E = edge_index.shape[1]
    H, HD = att_src1.shape
    D1 = H * HD
    NC = W2.shape[1]

    npad = _NSUB * _CH
    NP = -(-N // npad) * npad
    epad = _NCORE * _NSUB * _CH
    EP = -(-E // epad) * epad

    xp = jnp.zeros((NP, F), jnp.float32).at[:N].set(x)
    src = edge_index[0]
    dst = edge_index[1]
    fill = jnp.full((EP - E,), N, jnp.int32)
    srcp = jnp.concatenate([src, fill]).reshape(EP // _CH, _CH)
    dstp = jnp.concatenate([dst, fill]).reshape(EP // _CH, _CH)

    eyeH = jnp.eye(H, 16, dtype=jnp.float32)
    A1s = (att_src1[:, :, None] * eyeH[:, None, :]).reshape(D1, 16)
    A1d = (att_dst1[:, :, None] * eyeH[:, None, :]).reshape(D1, 16)
    R1 = jnp.broadcast_to(jnp.eye(16, H, dtype=jnp.float32)[:, :, None],
                          (16, H, HD)).reshape(16, D1)
    A2s = jnp.pad(att_src2.T, ((0, 0), (0, 15)))
    A2d = jnp.pad(att_dst2.T, ((0, 0), (0, 15)))
    R2 = jnp.zeros((16, NC), jnp.float32).at[0].set(1.0)
    b1r = b1.reshape(1, D1)
    b2r = b2.reshape(1, NC)

    h1, tas1, tad1, c1m = _prep(xp, W1, A1s, A1d)
    c1v = c1m[0, :16]
    nump1, denp1 = _make_edge_fn(NP, EP, H, D1)(srcp, dstp, h1, tas1, tad1,
                                                c1v)
    out1f, h2, tas2, tad2, c2m = _mid(nump1[0], nump1[1], denp1[0], denp1[1],
                                      R1, b1r, W2, A2s, A2d)
    c2v = c2m[0, :16]
    nump2, denp2 = _make_edge_fn(NP, EP, 1, NC)(srcp, dstp, h2, tas2, tad2,
                                                c2v)
    lsm = _fin(nump2[0], nump2[1], denp2[0], denp2[1], R2, b2r)[0]
    return (lsm[:N], out1f[:N])


# speedup vs baseline: 26.6175x; 26.6175x over previous
import jax
import jax.numpy as jnp
from jax import lax
from jax.experimental import pallas as pl
from jax.experimental.pallas import tpu as pltpu
from jax.experimental.pallas import tpu_sc as plsc

_CH = 64
_NSUB = 16
_NCORE = 2
_BLK = 1024


def _prep_body(x_ref, w_ref, a0_ref, d0_ref, a1_ref, d1_ref, c0_ref, c1_ref,
               ms_ref):
    i = pl.program_id(0)
    g = pl.num_programs(0)
    t = jnp.dot(x_ref[...], w_ref[...], preferred_element_type=jnp.float32)
    a0_ref[...] = t[:, 0:128]
    d0_ref[...] = t[:, 128:256]
    a1_ref[...] = t[:, 256:384]
    d1_ref[...] = t[:, 384:512]
    for j, (lo, hi) in enumerate(((64, 68), (192, 196), (320, 324),
                                  (448, 452))):
        prev = jnp.where(i == 0, -jnp.inf, ms_ref[j])
        ms_ref[j] = jnp.maximum(prev, jnp.max(t[:, lo:hi]))

    @pl.when(i == g - 1)
    def _():
        c0_ref[...] = jnp.full((1, 128), ms_ref[0] + ms_ref[1], jnp.float32)
        c1_ref[...] = jnp.full((1, 128), ms_ref[2] + ms_ref[3], jnp.float32)


def _prep(xp, Wcat):
    NP, F = xp.shape
    grid = (NP // _BLK,)
    tab = pl.BlockSpec((_BLK, 128), lambda i: (i, 0))
    tshape = jax.ShapeDtypeStruct((NP, 128), jnp.float32)
    cspec = pl.BlockSpec((1, 128), lambda i: (0, 0))
    cshape = jax.ShapeDtypeStruct((1, 128), jnp.float32)
    return pl.pallas_call(
        _prep_body,
        grid=grid,
        in_specs=[
            pl.BlockSpec((_BLK, F), lambda i: (i, 0)),
            pl.BlockSpec((F, 512), lambda i: (0, 0)),
        ],
        out_specs=[tab, tab, tab, tab, cspec, cspec],
        out_shape=[tshape, tshape, tshape, tshape, cshape, cshape],
        scratch_shapes=[pltpu.SMEM((4,), jnp.float32)],
    )(xp, Wcat)


def _make_edge_fn(NP, EP, nh, mw):
    nw = _NCORE * _NSUB
    chunks_pw = EP // (nw * _CH)
    rows_ps = NP // _NSUB
    zch = rows_ps // _CH
    mesh = plsc.VectorSubcoreMesh(core_axis_name="c", subcore_axis_name="s",
                                  num_cores=_NCORE, num_subcores=_NSUB)

    def body(src_ref, dst_ref, tabA_ref, tabD_ref, c_ref, accp_ref,
             acc, vsrc, vdst, rowsA, rowsD, cbuf, semA, semD):
        cid = lax.axis_index("c")
        sid = lax.axis_index("s")

        def zrow(i, _):
            for j in range(8):
                rowsA[i, pl.ds(j * 16, 16)] = jnp.zeros((16,), jnp.float32)
            return 0

        lax.fori_loop(0, _CH, zrow, 0)

        def zacc(k, _):
            r = sid * rows_ps + k * _CH
            pltpu.sync_copy(rowsA, acc.at[pl.ds(r, _CH)])
            return 0

        lax.fori_loop(0, zch, zacc, 0)
        pltpu.sync_copy(c_ref, cbuf)
        plsc.subcore_barrier()
        cvec = cbuf[...]
        wid = cid * _NSUB + sid

        def chunk(k, _):
            base = (wid * chunks_pw + k) * _CH
            pltpu.sync_copy(src_ref.at[pl.ds(base, _CH)], vsrc)
            pltpu.sync_copy(dst_ref.at[pl.ds(base, _CH)], vdst)
            cpa = pltpu.async_copy(tabA_ref.at[vsrc], rowsA, semA)
            cpd = pltpu.async_copy(tabD_ref.at[vdst], rowsD, semD)
            cpa.wait()
            cpd.wait()

            def erow(i, _):
                t = rowsA[i, pl.ds(64, 16)] + rowsD[i, pl.ds(64, 16)]
                alpha = jnp.maximum(t, 0.2 * t)
                e16 = jnp.exp(alpha - cvec)
                rowsA[i, pl.ds(64, 16)] = e16
                for hh in range(nh):
                    ev = e16[hh]
                    for j in range(mw // 16):
                        off = hh * mw + j * 16
                        rowsA[i, pl.ds(off, 16)] = rowsA[i, pl.ds(off, 16)] * ev
                return 0

            lax.fori_loop(0, _CH, erow, 0)
            pltpu.sync_copy(rowsA, acc.at[vdst], add=True)
            return 0

        lax.fori_loop(0, chunks_pw, chunk, 0)
        plsc.subcore_barrier()

        def wout(k, _):
            r = sid * rows_ps + k * _CH
            pltpu.sync_copy(acc.at[pl.ds(r, _CH)],
                            accp_ref.at[cid, pl.ds(r, _CH)])
            return 0

        lax.fori_loop(0, zch, wout, 0)

    return pl.kernel(
        body,
        out_type=jax.ShapeDtypeStruct((_NCORE, NP, 128), jnp.float32),
        mesh=mesh,
        scratch_types=[
            pltpu.VMEM_SHARED((NP, 128), jnp.float32),
            pltpu.VMEM((_CH,), jnp.int32),
            pltpu.VMEM((_CH,), jnp.int32),
            pltpu.VMEM((_CH, 128), jnp.float32),
            pltpu.VMEM((_CH, 128), jnp.float32),
            pltpu.VMEM((16,), jnp.float32),
            pltpu.SemaphoreType.DMA,
            pltpu.SemaphoreType.DMA,
        ],
    )


def _mid_body(p00_ref, p01_ref, p10_ref, p11_ref, rd_ref, b1_ref,
              wa_ref, wd_ref,
              out1_ref, tA2_ref, tD2_ref, c_ref, ms_ref):
    i = pl.program_id(0)
    g = pl.num_programs(0)
    q0 = p00_ref[...] + p01_ref[...]
    q1 = p10_ref[...] + p11_ref[...]
    den0 = jnp.dot(q0, rd_ref[...], preferred_element_type=jnp.float32)
    den1 = jnp.dot(q1, rd_ref[...], preferred_element_type=jnp.float32)
    o = jnp.concatenate(
        [q0[:, :64] / (den0 + 1e-16), q1[:, :64] / (den1 + 1e-16)], axis=1)
    o = o + b1_ref[...]
    out1_ref[...] = o
    gact = jnp.where(o > 0, o, jnp.exp(o) - 1.0)
    tA2 = jnp.dot(gact, wa_ref[...], preferred_element_type=jnp.float32)
    tD2 = jnp.dot(gact, wd_ref[...], preferred_element_type=jnp.float32)
    tA2_ref[...] = tA2
    tD2_ref[...] = tD2
    prev_s = jnp.where(i == 0, -jnp.inf, ms_ref[0])
    prev_d = jnp.where(i == 0, -jnp.inf, ms_ref[1])
    ms_ref[0] = jnp.maximum(prev_s, jnp.max(tA2[:, 64:65]))
    ms_ref[1] = jnp.maximum(prev_d, jnp.max(tD2[:, 64:65]))

    @pl.when(i == g - 1)
    def _():
        c_ref[...] = jnp.full((1, 128), ms_ref[0] + ms_ref[1], jnp.float32)


def _mid(p00, p01, p10, p11, Rden, b1r, WA2, WD2):
    NP = p00.shape[0]
    grid = (NP // _BLK,)
    blk = pl.BlockSpec((_BLK, 128), lambda i: (i, 0))
    shp = jax.ShapeDtypeStruct((NP, 128), jnp.float32)
    return pl.pallas_call(
        _mid_body,
        grid=grid,
        in_specs=[
            blk, blk, blk, blk,
            pl.BlockSpec((128, 64), lambda i: (0, 0)),
            pl.BlockSpec((1, 128), lambda i: (0, 0)),
            pl.BlockSpec((128, 128), lambda i: (0, 0)),
            pl.BlockSpec((128, 128), lambda i: (0, 0)),
        ],
        out_specs=[blk, blk, blk, pl.BlockSpec((1, 128), lambda i: (0, 0))],
        out_shape=[shp, shp, shp,
                   jax.ShapeDtypeStruct((1, 128), jnp.float32)],
        scratch_shapes=[pltpu.SMEM((2,), jnp.float32)],
    )(p00, p01, p10, p11, Rden, b1r, WA2, WD2)


def _fin_body(p0_ref, p1_ref, rd_ref, b2_ref, out_ref):
    q = p0_ref[...] + p1_ref[...]
    den = jnp.dot(q, rd_ref[...], preferred_element_type=jnp.float32)
    o2 = q[:, :64] / (den + 1e-16) + b2_ref[...]
    m = jnp.max(o2, axis=1, keepdims=True)
    ex = jnp.exp(o2 - m)
    lse = jnp.log(jnp.sum(ex, axis=1, keepdims=True))
    out_ref[...] = o2 - m - lse


def _fin(p0, p1, Rden2, b2r):
    NP = p0.shape[0]
    grid = (NP // _BLK,)
    blk = pl.BlockSpec((_BLK, 128), lambda i: (i, 0))
    return pl.pallas_call(
        _fin_body,
        grid=grid,
        in_specs=[
            blk, blk,
            pl.BlockSpec((128, 64), lambda i: (0, 0)),
            pl.BlockSpec((1, 64), lambda i: (0, 0)),
        ],
        out_specs=[pl.BlockSpec((_BLK, 64), lambda i: (i, 0))],
        out_shape=[jax.ShapeDtypeStruct((NP, 64), jnp.float32)],
    )(p0, p1, Rden2, b2r)


def kernel(x, edge_index, encoder_type, W1, att_src1, att_dst1, b1,
           W2, att_src2, att_dst2, b2):
    del encoder_type
    N, F = x.shape
    E = edge_index.shape[1]
    H, HD = att_src1.shape
    D1 = H * HD
    NC = W2.shape[1]

    npad = _NSUB * _CH
    NP = -(-N // npad) * npad
    epad = _NCORE * _NSUB * _CH
    EP = -(-E // epad) * epad

    xp = jnp.zeros((NP, F), jnp.float32).at[:N].set(x)
    fill = jnp.full((EP - E,), N, jnp.int32)
    srcp = jnp.concatenate([edge_index[0], fill])
    dstp = jnp.concatenate([edge_index[1], fill])

    eyeH = jnp.eye(H, 8, dtype=jnp.float32)
    Ablk_s = (att_src1[:, :, None] * eyeH[:, None, :]).reshape(D1, 8)
    Ablk_d = (att_dst1[:, :, None] * eyeH[:, None, :]).reshape(D1, 8)
    w1s = W1 @ Ablk_s
    w1d = W1 @ Ablk_d
    z56 = jnp.zeros((F, 56), jnp.float32)
    z60 = jnp.zeros((F, 60), jnp.float32)
    z64 = jnp.zeros((F, 64), jnp.float32)
    tA0 = jnp.concatenate([W1[:, 0:64], w1s[:, 0:4], z60], axis=1)
    tD0 = jnp.concatenate([z64, w1d[:, 0:4], z60], axis=1)
    tA1 = jnp.concatenate([W1[:, 64:128], w1s[:, 4:8], z60], axis=1)
    tD1 = jnp.concatenate([z64, w1d[:, 4:8], z60], axis=1)
    Wcat = jnp.concatenate([tA0, tD0, tA1, tD1], axis=1)

    w2s = W2 @ att_src2.T
    w2d = W2 @ att_dst2.T
    z63 = jnp.zeros((D1, 63), jnp.float32)
    WA2 = jnp.concatenate([W2, w2s, z63], axis=1)
    WD2 = jnp.concatenate([jnp.zeros((D1, 64), jnp.float32), w2d, z63],
                          axis=1)

    hd4 = jnp.eye(4, dtype=jnp.float32)
    Rden = jnp.zeros((128, 64), jnp.float32).at[64:68].set(
        jnp.repeat(hd4, HD, axis=1))
    Rden2 = jnp.zeros((128, 64), jnp.float32).at[64].set(1.0)
    b1r = b1.reshape(1, D1)
    b2r = b2.reshape(1, NC)

    tabA0, tabD0, tabA1, tabD1, c0m, c1m = _prep(xp, Wcat)
    edge4 = _make_edge_fn(NP, EP, 4, HD)
    p0 = edge4(srcp, dstp, tabA0, tabD0, c0m[0, :16])
    p1 = edge4(srcp, dstp, tabA1, tabD1, c1m[0, :16])
    out1f, tabA2, tabD2, c2m = _mid(p0[0], p0[1], p1[0], p1[1],
                                    Rden, b1r, WA2, WD2)
    p2 = _make_edge_fn(NP, EP, 1, NC)(srcp, dstp, tabA2, tabD2, c2m[0, :16])
    lsm = _fin(p2[0], p2[1], Rden2, b2r)[0]
    return (lsm[:N], out1f[:N])


# speedup vs baseline: 29.2860x; 1.1003x over previous
import jax
import jax.numpy as jnp
from jax import lax
from jax.experimental import pallas as pl
from jax.experimental.pallas import tpu as pltpu
from jax.experimental.pallas import tpu_sc as plsc

_CH = 48
_NSUB = 16
_NCORE = 2
_BLK = 1024


def _prep_body(x_ref, w_ref, a0_ref, d0_ref, a1_ref, d1_ref, c0_ref, c1_ref,
               ms_ref):
    i = pl.program_id(0)
    g = pl.num_programs(0)
    t = jnp.dot(x_ref[...], w_ref[...], preferred_element_type=jnp.float32)
    a0_ref[...] = t[:, 0:128]
    d0_ref[...] = t[:, 128:256]
    a1_ref[...] = t[:, 256:384]
    d1_ref[...] = t[:, 384:512]
    for j, (lo, hi) in enumerate(((64, 68), (192, 196), (320, 324),
                                  (448, 452))):
        prev = jnp.where(i == 0, -jnp.inf, ms_ref[j])
        ms_ref[j] = jnp.maximum(prev, jnp.max(t[:, lo:hi]))

    @pl.when(i == g - 1)
    def _():
        c0_ref[...] = jnp.full((1, 128), ms_ref[0] + ms_ref[1], jnp.float32)
        c1_ref[...] = jnp.full((1, 128), ms_ref[2] + ms_ref[3], jnp.float32)


def _prep(xp, Wcat):
    NP, F = xp.shape
    grid = (NP // _BLK,)
    tab = pl.BlockSpec((_BLK, 128), lambda i: (i, 0))
    tshape = jax.ShapeDtypeStruct((NP, 128), jnp.float32)
    cspec = pl.BlockSpec((1, 128), lambda i: (0, 0))
    cshape = jax.ShapeDtypeStruct((1, 128), jnp.float32)
    return pl.pallas_call(
        _prep_body,
        grid=grid,
        in_specs=[
            pl.BlockSpec((_BLK, F), lambda i: (i, 0)),
            pl.BlockSpec((F, 512), lambda i: (0, 0)),
        ],
        out_specs=[tab, tab, tab, tab, cspec, cspec],
        out_shape=[tshape, tshape, tshape, tshape, cshape, cshape],
        scratch_shapes=[pltpu.SMEM((4,), jnp.float32)],
    )(xp, Wcat)


def _make_edge_fn(NP, EP, nh, mw):
    nw = _NCORE * _NSUB
    chunks_pw = EP // (nw * _CH)
    npairs = chunks_pw // 2
    assert chunks_pw % 2 == 0
    rows_ps = NP // _NSUB
    zrows = _CH
    assert rows_ps % zrows == 0
    zch = rows_ps // zrows
    mesh = plsc.VectorSubcoreMesh(core_axis_name="c", subcore_axis_name="s",
                                  num_cores=_NCORE, num_subcores=_NSUB)

    def body(src_ref, dst_ref, tabA_ref, tabD_ref, c_ref, accp_ref,
             acc, vsrc0, vsrc1, vdst0, vdst1, wdst0, wdst1,
             rowsA0, rowsA1, rowsD0, rowsD1, wbuf0, wbuf1, cbuf,
             semA0, semA1, semD0, semD1, semW0, semW1):
        cid = lax.axis_index("c")
        sid = lax.axis_index("s")
        vsrc = (vsrc0, vsrc1)
        vdst = (vdst0, vdst1)
        wdst = (wdst0, wdst1)
        rowsA = (rowsA0, rowsA1)
        rowsD = (rowsD0, rowsD1)
        wbuf = (wbuf0, wbuf1)
        semA = (semA0, semA1)
        semD = (semD0, semD1)
        semW = (semW0, semW1)

        def zrow(i, _):
            for j in range(8):
                wbuf0[i, pl.ds(j * 16, 16)] = jnp.zeros((16,), jnp.float32)
                wbuf1[i, pl.ds(j * 16, 16)] = jnp.zeros((16,), jnp.float32)
            return 0

        lax.fori_loop(0, _CH, zrow, 0)
        for j in range(_CH // 16):
            wdst0[pl.ds(j * 16, 16)] = jnp.zeros((16,), jnp.int32)
            wdst1[pl.ds(j * 16, 16)] = jnp.zeros((16,), jnp.int32)

        def zacc(k, _):
            r = sid * rows_ps + k * zrows
            pltpu.sync_copy(wbuf0.at[pl.ds(0, zrows)], acc.at[pl.ds(r, zrows)])
            return 0

        lax.fori_loop(0, zch, zacc, 0)
        pltpu.sync_copy(c_ref, cbuf)
        plsc.subcore_barrier()
        cvec = cbuf[...]
        wid = cid * _NSUB + sid
        wbase = wid * chunks_pw

        pltpu.async_copy(wbuf0, acc.at[wdst0], semW0, add=True)
        pltpu.async_copy(wbuf1, acc.at[wdst1], semW1, add=True)
        pltpu.sync_copy(src_ref.at[pl.ds(wbase * _CH, _CH)], vsrc0)
        pltpu.sync_copy(dst_ref.at[pl.ds(wbase * _CH, _CH)], vdst0)
        pltpu.async_copy(tabA_ref.at[vsrc0], rowsA0, semA0)
        pltpu.async_copy(tabD_ref.at[vdst0], rowsD0, semD0)

        def pair(p, _):
            for b in range(2):
                nb = 1 - b
                g = 2 * p + b
                gn = lax.rem(g + 1, chunks_pw)
                base = (wbase + gn) * _CH
                pltpu.sync_copy(src_ref.at[pl.ds(base, _CH)], vsrc[nb])
                pltpu.sync_copy(dst_ref.at[pl.ds(base, _CH)], vdst[nb])
                pltpu.async_copy(tabA_ref.at[vsrc[nb]], rowsA[nb], semA[nb])
                pltpu.async_copy(tabD_ref.at[vdst[nb]], rowsD[nb], semD[nb])
                pltpu.make_async_copy(tabA_ref.at[vsrc[b]], rowsA[b],
                                      semA[b]).wait()
                pltpu.make_async_copy(tabD_ref.at[vdst[b]], rowsD[b],
                                      semD[b]).wait()
                pltpu.make_async_copy(wbuf[b], acc.at[wdst[b]],
                                      semW[b]).wait()
                for j in range(_CH // 16):
                    wdst[b][pl.ds(j * 16, 16)] = vdst[b][pl.ds(j * 16, 16)]

                def erow(i, _):
                    t = (rowsA[b][i, pl.ds(64, 16)]
                         + rowsD[b][i, pl.ds(64, 16)])
                    alpha = jnp.maximum(t, 0.2 * t)
                    e16 = jnp.exp(alpha - cvec)
                    wbuf[b][i, pl.ds(64, 16)] = e16
                    for hh in range(nh):
                        ev = e16[hh]
                        for j2 in range(mw // 16):
                            off = hh * mw + j2 * 16
                            wbuf[b][i, pl.ds(off, 16)] = (
                                rowsA[b][i, pl.ds(off, 16)] * ev)
                    return 0

                lax.fori_loop(0, _CH, erow, 0, unroll=2)
                pltpu.async_copy(wbuf[b], acc.at[wdst[b]], semW[b], add=True)
            return 0

        lax.fori_loop(0, npairs, pair, 0)
        pltpu.make_async_copy(tabA_ref.at[vsrc0], rowsA0, semA0).wait()
        pltpu.make_async_copy(tabD_ref.at[vdst0], rowsD0, semD0).wait()
        pltpu.make_async_copy(wbuf0, acc.at[wdst0], semW0).wait()
        pltpu.make_async_copy(wbuf1, acc.at[wdst1], semW1).wait()
        plsc.subcore_barrier()

        def wout(k, _):
            r = sid * rows_ps + k * zrows
            pltpu.sync_copy(acc.at[pl.ds(r, zrows)],
                            accp_ref.at[cid, pl.ds(r, zrows)])
            return 0

        lax.fori_loop(0, zch, wout, 0)

    return pl.kernel(
        body,
        out_type=jax.ShapeDtypeStruct((_NCORE, NP, 128), jnp.float32),
        mesh=mesh,
        scratch_types=[
            pltpu.VMEM_SHARED((NP, 128), jnp.float32),
            pltpu.VMEM((_CH,), jnp.int32),
            pltpu.VMEM((_CH,), jnp.int32),
            pltpu.VMEM((_CH,), jnp.int32),
            pltpu.VMEM((_CH,), jnp.int32),
            pltpu.VMEM((_CH,), jnp.int32),
            pltpu.VMEM((_CH,), jnp.int32),
            pltpu.VMEM((_CH, 128), jnp.float32),
            pltpu.VMEM((_CH, 128), jnp.float32),
            pltpu.VMEM((_CH, 128), jnp.float32),
            pltpu.VMEM((_CH, 128), jnp.float32),
            pltpu.VMEM((_CH, 128), jnp.float32),
            pltpu.VMEM((_CH, 128), jnp.float32),
            pltpu.VMEM((16,), jnp.float32),
            pltpu.SemaphoreType.DMA,
            pltpu.SemaphoreType.DMA,
            pltpu.SemaphoreType.DMA,
            pltpu.SemaphoreType.DMA,
            pltpu.SemaphoreType.DMA,
            pltpu.SemaphoreType.DMA,
        ],
    )


def _mid_body(p00_ref, p01_ref, p10_ref, p11_ref, rd_ref, b1_ref,
              wa_ref, wd_ref,
              out1_ref, tA2_ref, tD2_ref, c_ref, ms_ref):
    i = pl.program_id(0)
    g = pl.num_programs(0)
    q0 = p00_ref[...] + p01_ref[...]
    q1 = p10_ref[...] + p11_ref[...]
    den0 = jnp.dot(q0, rd_ref[...], preferred_element_type=jnp.float32)
    den1 = jnp.dot(q1, rd_ref[...], preferred_element_type=jnp.float32)
    o = jnp.concatenate(
        [q0[:, :64] / (den0 + 1e-16), q1[:, :64] / (den1 + 1e-16)], axis=1)
    o = o + b1_ref[...]
    out1_ref[...] = o
    gact = jnp.where(o > 0, o, jnp.exp(o) - 1.0)
    tA2 = jnp.dot(gact, wa_ref[...], preferred_element_type=jnp.float32)
    tD2 = jnp.dot(gact, wd_ref[...], preferred_element_type=jnp.float32)
    tA2_ref[...] = tA2
    tD2_ref[...] = tD2
    prev_s = jnp.where(i == 0, -jnp.inf, ms_ref[0])
    prev_d = jnp.where(i == 0, -jnp.inf, ms_ref[1])
    ms_ref[0] = jnp.maximum(prev_s, jnp.max(tA2[:, 64:65]))
    ms_ref[1] = jnp.maximum(prev_d, jnp.max(tD2[:, 64:65]))

    @pl.when(i == g - 1)
    def _():
        c_ref[...] = jnp.full((1, 128), ms_ref[0] + ms_ref[1], jnp.float32)


def _mid(p00, p01, p10, p11, Rden, b1r, WA2, WD2):
    NP = p00.shape[0]
    grid = (NP // _BLK,)
    blk = pl.BlockSpec((_BLK, 128), lambda i: (i, 0))
    shp = jax.ShapeDtypeStruct((NP, 128), jnp.float32)
    return pl.pallas_call(
        _mid_body,
        grid=grid,
        in_specs=[
            blk, blk, blk, blk,
            pl.BlockSpec((128, 64), lambda i: (0, 0)),
            pl.BlockSpec((1, 128), lambda i: (0, 0)),
            pl.BlockSpec((128, 128), lambda i: (0, 0)),
            pl.BlockSpec((128, 128), lambda i: (0, 0)),
        ],
        out_specs=[blk, blk, blk, pl.BlockSpec((1, 128), lambda i: (0, 0))],
        out_shape=[shp, shp, shp,
                   jax.ShapeDtypeStruct((1, 128), jnp.float32)],
        scratch_shapes=[pltpu.SMEM((2,), jnp.float32)],
    )(p00, p01, p10, p11, Rden, b1r, WA2, WD2)


def _fin_body(p0_ref, p1_ref, rd_ref, b2_ref, out_ref):
    q = p0_ref[...] + p1_ref[...]
    den = jnp.dot(q, rd_ref[...], preferred_element_type=jnp.float32)
    o2 = q[:, :64] / (den + 1e-16) + b2_ref[...]
    m = jnp.max(o2, axis=1, keepdims=True)
    ex = jnp.exp(o2 - m)
    lse = jnp.log(jnp.sum(ex, axis=1, keepdims=True))
    out_ref[...] = o2 - m - lse


def _fin(p0, p1, Rden2, b2r):
    NP = p0.shape[0]
    grid = (NP // _BLK,)
    blk = pl.BlockSpec((_BLK, 128), lambda i: (i, 0))
    return pl.pallas_call(
        _fin_body,
        grid=grid,
        in_specs=[
            blk, blk,
            pl.BlockSpec((128, 64), lambda i: (0, 0)),
            pl.BlockSpec((1, 64), lambda i: (0, 0)),
        ],
        out_specs=[pl.BlockSpec((_BLK, 64), lambda i: (i, 0))],
        out_shape=[jax.ShapeDtypeStruct((NP, 64), jnp.float32)],
    )(p0, p1, Rden2, b2r)


def kernel(x, edge_index, encoder_type, W1, att_src1, att_dst1, b1,
           W2, att_src2, att_dst2, b2):
    del encoder_type
    N, F = x.shape
    E = edge_index.shape[1]
    H, HD = att_src1.shape
    D1 = H * HD
    NC = W2.shape[1]

    npad = _NSUB * _CH
    NP = -(-N // npad) * npad
    epad = 2 * _NCORE * _NSUB * _CH
    EP = -(-E // epad) * epad

    xp = jnp.zeros((NP, F), jnp.float32).at[:N].set(x)
    fill = jnp.full((EP - E,), N, jnp.int32)
    srcp = jnp.concatenate([edge_index[0], fill])
    dstp = jnp.concatenate([edge_index[1], fill])

    eyeH = jnp.eye(H, 8, dtype=jnp.float32)
    Ablk_s = (att_src1[:, :, None] * eyeH[:, None, :]).reshape(D1, 8)
    Ablk_d = (att_dst1[:, :, None] * eyeH[:, None, :]).reshape(D1, 8)
    w1s = W1 @ Ablk_s
    w1d = W1 @ Ablk_d
    z56 = jnp.zeros((F, 56), jnp.float32)
    z60 = jnp.zeros((F, 60), jnp.float32)
    z64 = jnp.zeros((F, 64), jnp.float32)
    tA0 = jnp.concatenate([W1[:, 0:64], w1s[:, 0:4], z60], axis=1)
    tD0 = jnp.concatenate([z64, w1d[:, 0:4], z60], axis=1)
    tA1 = jnp.concatenate([W1[:, 64:128], w1s[:, 4:8], z60], axis=1)
    tD1 = jnp.concatenate([z64, w1d[:, 4:8], z60], axis=1)
    Wcat = jnp.concatenate([tA0, tD0, tA1, tD1], axis=1)

    w2s = W2 @ att_src2.T
    w2d = W2 @ att_dst2.T
    z63 = jnp.zeros((D1, 63), jnp.float32)
    WA2 = jnp.concatenate([W2, w2s, z63], axis=1)
    WD2 = jnp.concatenate([jnp.zeros((D1, 64), jnp.float32), w2d, z63],
                          axis=1)

    hd4 = jnp.eye(4, dtype=jnp.float32)
    Rden = jnp.zeros((128, 64), jnp.float32).at[64:68].set(
        jnp.repeat(hd4, HD, axis=1))
    Rden2 = jnp.zeros((128, 64), jnp.float32).at[64].set(1.0)
    b1r = b1.reshape(1, D1)
    b2r = b2.reshape(1, NC)

    tabA0, tabD0, tabA1, tabD1, c0m, c1m = _prep(xp, Wcat)
    edge4 = _make_edge_fn(NP, EP, 4, HD)
    p0 = edge4(srcp, dstp, tabA0, tabD0, c0m[0, :16])
    p1 = edge4(srcp, dstp, tabA1, tabD1, c1m[0, :16])
    out1f, tabA2, tabD2, c2m = _mid(p0[0], p0[1], p1[0], p1[1],
                                    Rden, b1r, WA2, WD2)
    p2 = _make_edge_fn(NP, EP, 1, NC)(srcp, dstp, tabA2, tabD2, c2m[0, :16])
    lsm = _fin(p2[0], p2[1], Rden2, b2r)[0]
    return (lsm[:N], out1f[:N])


# speedup vs baseline: 37.0148x; 1.2639x over previous
import jax
import jax.numpy as jnp
from jax import lax
from jax.experimental import pallas as pl
from jax.experimental.pallas import tpu as pltpu
from jax.experimental.pallas import tpu_sc as plsc

_CH = 48
_NSUB = 16
_NCORE = 2
_BLK = 1024


def _prep_body(x_ref, w_ref, a0_ref, d0_ref, a1_ref, d1_ref, c0_ref, c1_ref,
               ms_ref):
    i = pl.program_id(0)
    g = pl.num_programs(0)
    t = jnp.dot(x_ref[...], w_ref[...], preferred_element_type=jnp.float32)
    a0_ref[...] = t[:, 0:128]
    d0_ref[...] = t[:, 128:256]
    a1_ref[...] = t[:, 256:384]
    d1_ref[...] = t[:, 384:512]
    for j, (lo, hi) in enumerate(((64, 68), (192, 196), (320, 324),
                                  (448, 452))):
        prev = jnp.where(i == 0, -jnp.inf, ms_ref[j])
        ms_ref[j] = jnp.maximum(prev, jnp.max(t[:, lo:hi]))

    @pl.when(i == g - 1)
    def _():
        c0_ref[...] = jnp.full((1, 128), ms_ref[0] + ms_ref[1], jnp.float32)
        c1_ref[...] = jnp.full((1, 128), ms_ref[2] + ms_ref[3], jnp.float32)


def _prep(xp, Wcat):
    NP, F = xp.shape
    grid = (NP // _BLK,)
    tab = pl.BlockSpec((_BLK, 128), lambda i: (i, 0))
    tshape = jax.ShapeDtypeStruct((NP, 128), jnp.float32)
    cspec = pl.BlockSpec((1, 128), lambda i: (0, 0))
    cshape = jax.ShapeDtypeStruct((1, 128), jnp.float32)
    return pl.pallas_call(
        _prep_body,
        grid=grid,
        in_specs=[
            pl.BlockSpec((_BLK, F), lambda i: (i, 0)),
            pl.BlockSpec((F, 512), lambda i: (0, 0)),
        ],
        out_specs=[tab, tab, tab, tab, cspec, cspec],
        out_shape=[tshape, tshape, tshape, tshape, cshape, cshape],
        scratch_shapes=[pltpu.SMEM((4,), jnp.float32)],
    )(xp, Wcat)


def _make_edge_fn(NP, EP, nh, mw):
    nw = _NCORE * _NSUB
    chunks_pw = EP // (nw * _CH)
    npairs = chunks_pw // 2
    assert chunks_pw % 2 == 0
    rows_ps = NP // _NSUB
    zrows = _CH
    assert rows_ps % zrows == 0
    zch = rows_ps // zrows
    mesh = plsc.VectorSubcoreMesh(core_axis_name="c", subcore_axis_name="s",
                                  num_cores=_NCORE, num_subcores=_NSUB)

    def body(src_ref, dst_ref, tabA_ref, tabD_ref, c_ref, accp_ref,
             acc, vsrc0, vsrc1, vdst0, vdst1, wdst0, wdst1,
             rowsA0, rowsA1, rowsD0, rowsD1, wbuf0, wbuf1, cbuf,
             semA0, semA1, semD0, semD1, semW0, semW1,
             semIS0, semIS1, semID0, semID1):
        cid = lax.axis_index("c")
        sid = lax.axis_index("s")
        vsrc = (vsrc0, vsrc1)
        vdst = (vdst0, vdst1)
        wdst = (wdst0, wdst1)
        rowsA = (rowsA0, rowsA1)
        rowsD = (rowsD0, rowsD1)
        wbuf = (wbuf0, wbuf1)
        semA = (semA0, semA1)
        semD = (semD0, semD1)
        semW = (semW0, semW1)
        semIS = (semIS0, semIS1)
        semID = (semID0, semID1)

        def zrow(i, _):
            for j in range(8):
                wbuf0[i, pl.ds(j * 16, 16)] = jnp.zeros((16,), jnp.float32)
                wbuf1[i, pl.ds(j * 16, 16)] = jnp.zeros((16,), jnp.float32)
            return 0

        lax.fori_loop(0, _CH, zrow, 0)
        for j in range(_CH // 16):
            wdst0[pl.ds(j * 16, 16)] = jnp.zeros((16,), jnp.int32)
            wdst1[pl.ds(j * 16, 16)] = jnp.zeros((16,), jnp.int32)

        def zacc(k, _):
            r = sid * rows_ps + k * zrows
            pltpu.sync_copy(wbuf0.at[pl.ds(0, zrows)], acc.at[pl.ds(r, zrows)])
            return 0

        lax.fori_loop(0, zch, zacc, 0)
        pltpu.sync_copy(c_ref, cbuf)
        plsc.subcore_barrier()
        cvec = cbuf[...]
        wid = cid * _NSUB + sid
        wbase = wid * chunks_pw

        pltpu.async_copy(wbuf0, acc.at[wdst0], semW0, add=True)
        pltpu.async_copy(wbuf1, acc.at[wdst1], semW1, add=True)
        pltpu.sync_copy(src_ref.at[pl.ds(wbase * _CH, _CH)], vsrc0)
        pltpu.sync_copy(dst_ref.at[pl.ds(wbase * _CH, _CH)], vdst0)
        pltpu.async_copy(src_ref.at[pl.ds((wbase + 1) * _CH, _CH)], vsrc1,
                         semIS1)
        pltpu.async_copy(dst_ref.at[pl.ds((wbase + 1) * _CH, _CH)], vdst1,
                         semID1)
        pltpu.async_copy(tabA_ref.at[vsrc0], rowsA0, semA0)
        pltpu.async_copy(tabD_ref.at[vdst0], rowsD0, semD0)

        def pair(p, _):
            for b in range(2):
                nb = 1 - b
                g = 2 * p + b
                gn = lax.rem(g + 1, chunks_pw)
                base = (wbase + gn) * _CH
                pltpu.make_async_copy(src_ref.at[pl.ds(base, _CH)],
                                      vsrc[nb], semIS[nb]).wait()
                pltpu.make_async_copy(dst_ref.at[pl.ds(base, _CH)],
                                      vdst[nb], semID[nb]).wait()
                pltpu.async_copy(tabA_ref.at[vsrc[nb]], rowsA[nb], semA[nb])
                pltpu.async_copy(tabD_ref.at[vdst[nb]], rowsD[nb], semD[nb])
                pltpu.make_async_copy(tabA_ref.at[vsrc[b]], rowsA[b],
                                      semA[b]).wait()
                pltpu.make_async_copy(tabD_ref.at[vdst[b]], rowsD[b],
                                      semD[b]).wait()
                pltpu.make_async_copy(wbuf[b], acc.at[wdst[b]],
                                      semW[b]).wait()
                for j in range(_CH // 16):
                    wdst[b][pl.ds(j * 16, 16)] = vdst[b][pl.ds(j * 16, 16)]
                g2 = lax.rem(g + 2, chunks_pw)
                base2 = (wbase + g2) * _CH
                pltpu.async_copy(src_ref.at[pl.ds(base2, _CH)], vsrc[b],
                                 semIS[b])
                pltpu.async_copy(dst_ref.at[pl.ds(base2, _CH)], vdst[b],
                                 semID[b])

                def erow(i, _):
                    t = (rowsA[b][i, pl.ds(64, 16)]
                         + rowsD[b][i, pl.ds(64, 16)])
                    alpha = jnp.maximum(t, 0.2 * t)
                    e16 = jnp.exp(alpha - cvec)
                    wbuf[b][i, pl.ds(64, 16)] = e16
                    for hh in range(nh):
                        ev = e16[hh]
                        for j2 in range(mw // 16):
                            off = hh * mw + j2 * 16
                            wbuf[b][i, pl.ds(off, 16)] = (
                                rowsA[b][i, pl.ds(off, 16)] * ev)
                    return 0

                lax.fori_loop(0, _CH, erow, 0, unroll=2)
                pltpu.async_copy(wbuf[b], acc.at[wdst[b]], semW[b], add=True)
            return 0

        lax.fori_loop(0, npairs, pair, 0)
        pltpu.make_async_copy(src_ref.at[pl.ds(wbase * _CH, _CH)], vsrc1,
                              semIS1).wait()
        pltpu.make_async_copy(dst_ref.at[pl.ds(wbase * _CH, _CH)], vdst1,
                              semID1).wait()
        pltpu.make_async_copy(tabA_ref.at[vsrc0], rowsA0, semA0).wait()
        pltpu.make_async_copy(tabD_ref.at[vdst0], rowsD0, semD0).wait()
        pltpu.make_async_copy(wbuf0, acc.at[wdst0], semW0).wait()
        pltpu.make_async_copy(wbuf1, acc.at[wdst1], semW1).wait()
        plsc.subcore_barrier()

        def wout(k, _):
            r = sid * rows_ps + k * zrows
            pltpu.sync_copy(acc.at[pl.ds(r, zrows)],
                            accp_ref.at[cid, pl.ds(r, zrows)])
            return 0

        lax.fori_loop(0, zch, wout, 0)

    return pl.kernel(
        body,
        out_type=jax.ShapeDtypeStruct((_NCORE, NP, 128), jnp.float32),
        mesh=mesh,
        scratch_types=[
            pltpu.VMEM_SHARED((NP, 128), jnp.float32),
            pltpu.VMEM((_CH,), jnp.int32),
            pltpu.VMEM((_CH,), jnp.int32),
            pltpu.VMEM((_CH,), jnp.int32),
            pltpu.VMEM((_CH,), jnp.int32),
            pltpu.VMEM((_CH,), jnp.int32),
            pltpu.VMEM((_CH,), jnp.int32),
            pltpu.VMEM((_CH, 128), jnp.float32),
            pltpu.VMEM((_CH, 128), jnp.float32),
            pltpu.VMEM((_CH, 128), jnp.float32),
            pltpu.VMEM((_CH, 128), jnp.float32),
            pltpu.VMEM((_CH, 128), jnp.float32),
            pltpu.VMEM((_CH, 128), jnp.float32),
            pltpu.VMEM((16,), jnp.float32),
            pltpu.SemaphoreType.DMA,
            pltpu.SemaphoreType.DMA,
            pltpu.SemaphoreType.DMA,
            pltpu.SemaphoreType.DMA,
            pltpu.SemaphoreType.DMA,
            pltpu.SemaphoreType.DMA,
            pltpu.SemaphoreType.DMA,
            pltpu.SemaphoreType.DMA,
            pltpu.SemaphoreType.DMA,
            pltpu.SemaphoreType.DMA,
        ],
    )


def _mid_body(p00_ref, p01_ref, p10_ref, p11_ref, rd_ref, b1_ref,
              wa_ref, wd_ref,
              out1_ref, tA2_ref, tD2_ref, c_ref, ms_ref):
    i = pl.program_id(0)
    g = pl.num_programs(0)
    q0 = p00_ref[...] + p01_ref[...]
    q1 = p10_ref[...] + p11_ref[...]
    den0 = jnp.dot(q0, rd_ref[...], preferred_element_type=jnp.float32)
    den1 = jnp.dot(q1, rd_ref[...], preferred_element_type=jnp.float32)
    o = jnp.concatenate(
        [q0[:, :64] / (den0 + 1e-16), q1[:, :64] / (den1 + 1e-16)], axis=1)
    o = o + b1_ref[...]
    out1_ref[...] = o
    gact = jnp.where(o > 0, o, jnp.exp(o) - 1.0)
    tA2 = jnp.dot(gact, wa_ref[...], preferred_element_type=jnp.float32)
    tD2 = jnp.dot(gact, wd_ref[...], preferred_element_type=jnp.float32)
    tA2_ref[...] = tA2
    tD2_ref[...] = tD2
    prev_s = jnp.where(i == 0, -jnp.inf, ms_ref[0])
    prev_d = jnp.where(i == 0, -jnp.inf, ms_ref[1])
    ms_ref[0] = jnp.maximum(prev_s, jnp.max(tA2[:, 64:65]))
    ms_ref[1] = jnp.maximum(prev_d, jnp.max(tD2[:, 64:65]))

    @pl.when(i == g - 1)
    def _():
        c_ref[...] = jnp.full((1, 128), ms_ref[0] + ms_ref[1], jnp.float32)


def _mid(p00, p01, p10, p11, Rden, b1r, WA2, WD2):
    NP = p00.shape[0]
    grid = (NP // _BLK,)
    blk = pl.BlockSpec((_BLK, 128), lambda i: (i, 0))
    shp = jax.ShapeDtypeStruct((NP, 128), jnp.float32)
    return pl.pallas_call(
        _mid_body,
        grid=grid,
        in_specs=[
            blk, blk, blk, blk,
            pl.BlockSpec((128, 64), lambda i: (0, 0)),
            pl.BlockSpec((1, 128), lambda i: (0, 0)),
            pl.BlockSpec((128, 128), lambda i: (0, 0)),
            pl.BlockSpec((128, 128), lambda i: (0, 0)),
        ],
        out_specs=[blk, blk, blk, pl.BlockSpec((1, 128), lambda i: (0, 0))],
        out_shape=[shp, shp, shp,
                   jax.ShapeDtypeStruct((1, 128), jnp.float32)],
        scratch_shapes=[pltpu.SMEM((2,), jnp.float32)],
    )(p00, p01, p10, p11, Rden, b1r, WA2, WD2)


def _fin_body(p0_ref, p1_ref, rd_ref, b2_ref, out_ref):
    q = p0_ref[...] + p1_ref[...]
    den = jnp.dot(q, rd_ref[...], preferred_element_type=jnp.float32)
    o2 = q[:, :64] / (den + 1e-16) + b2_ref[...]
    m = jnp.max(o2, axis=1, keepdims=True)
    ex = jnp.exp(o2 - m)
    lse = jnp.log(jnp.sum(ex, axis=1, keepdims=True))
    out_ref[...] = o2 - m - lse


def _fin(p0, p1, Rden2, b2r):
    NP = p0.shape[0]
    grid = (NP // _BLK,)
    blk = pl.BlockSpec((_BLK, 128), lambda i: (i, 0))
    return pl.pallas_call(
        _fin_body,
        grid=grid,
        in_specs=[
            blk, blk,
            pl.BlockSpec((128, 64), lambda i: (0, 0)),
            pl.BlockSpec((1, 64), lambda i: (0, 0)),
        ],
        out_specs=[pl.BlockSpec((_BLK, 64), lambda i: (i, 0))],
        out_shape=[jax.ShapeDtypeStruct((NP, 64), jnp.float32)],
    )(p0, p1, Rden2, b2r)


def kernel(x, edge_index, encoder_type, W1, att_src1, att_dst1, b1,
           W2, att_src2, att_dst2, b2):
    del encoder_type
    N, F = x.shape
    E = edge_index.shape[1]
    H, HD = att_src1.shape
    D1 = H * HD
    NC = W2.shape[1]

    npad = _NSUB * _CH
    NP = -(-N // npad) * npad
    epad = 2 * _NCORE * _NSUB * _CH
    EP = -(-E // epad) * epad

    xp = jnp.zeros((NP, F), jnp.float32).at[:N].set(x)
    fill = jnp.full((EP - E,), N, jnp.int32)
    srcp = jnp.concatenate([edge_index[0], fill])
    dstp = jnp.concatenate([edge_index[1], fill])

    eyeH = jnp.eye(H, 8, dtype=jnp.float32)
    Ablk_s = (att_src1[:, :, None] * eyeH[:, None, :]).reshape(D1, 8)
    Ablk_d = (att_dst1[:, :, None] * eyeH[:, None, :]).reshape(D1, 8)
    w1s = W1 @ Ablk_s
    w1d = W1 @ Ablk_d
    z56 = jnp.zeros((F, 56), jnp.float32)
    z60 = jnp.zeros((F, 60), jnp.float32)
    z64 = jnp.zeros((F, 64), jnp.float32)
    tA0 = jnp.concatenate([W1[:, 0:64], w1s[:, 0:4], z60], axis=1)
    tD0 = jnp.concatenate([z64, w1d[:, 0:4], z60], axis=1)
    tA1 = jnp.concatenate([W1[:, 64:128], w1s[:, 4:8], z60], axis=1)
    tD1 = jnp.concatenate([z64, w1d[:, 4:8], z60], axis=1)
    Wcat = jnp.concatenate([tA0, tD0, tA1, tD1], axis=1)

    w2s = W2 @ att_src2.T
    w2d = W2 @ att_dst2.T
    z63 = jnp.zeros((D1, 63), jnp.float32)
    WA2 = jnp.concatenate([W2, w2s, z63], axis=1)
    WD2 = jnp.concatenate([jnp.zeros((D1, 64), jnp.float32), w2d, z63],
                          axis=1)

    hd4 = jnp.eye(4, dtype=jnp.float32)
    Rden = jnp.zeros((128, 64), jnp.float32).at[64:68].set(
        jnp.repeat(hd4, HD, axis=1))
    Rden2 = jnp.zeros((128, 64), jnp.float32).at[64].set(1.0)
    b1r = b1.reshape(1, D1)
    b2r = b2.reshape(1, NC)

    tabA0, tabD0, tabA1, tabD1, c0m, c1m = _prep(xp, Wcat)
    edge4 = _make_edge_fn(NP, EP, 4, HD)
    p0 = edge4(srcp, dstp, tabA0, tabD0, c0m[0, :16])
    p1 = edge4(srcp, dstp, tabA1, tabD1, c1m[0, :16])
    out1f, tabA2, tabD2, c2m = _mid(p0[0], p0[1], p1[0], p1[1],
                                    Rden, b1r, WA2, WD2)
    p2 = _make_edge_fn(NP, EP, 1, NC)(srcp, dstp, tabA2, tabD2, c2m[0, :16])
    lsm = _fin(p2[0], p2[1], Rden2, b2r)[0]
    return (lsm[:N], out1f[:N])


# speedup vs baseline: 37.1457x; 1.0035x over previous
import jax
import jax.numpy as jnp
from jax import lax
from jax.experimental import pallas as pl
from jax.experimental.pallas import tpu as pltpu
from jax.experimental.pallas import tpu_sc as plsc

_CH = 48
_NSUB = 16
_NCORE = 2
_BLK = 1024


def _prep_body(x_ref, w_ref, a0_ref, d0_ref, a1_ref, d1_ref, c0_ref, c1_ref,
               ms_ref):
    i = pl.program_id(0)
    g = pl.num_programs(0)
    t = jnp.dot(x_ref[...], w_ref[...], preferred_element_type=jnp.float32)
    a0_ref[...] = t[:, 0:128]
    d0_ref[...] = t[:, 128:256]
    a1_ref[...] = t[:, 256:384]
    d1_ref[...] = t[:, 384:512]
    for j, (lo, hi) in enumerate(((64, 68), (192, 196), (320, 324),
                                  (448, 452))):
        prev = jnp.where(i == 0, -jnp.inf, ms_ref[j])
        ms_ref[j] = jnp.maximum(prev, jnp.max(t[:, lo:hi]))

    @pl.when(i == g - 1)
    def _():
        c0_ref[...] = jnp.full((1, 128), ms_ref[0] + ms_ref[1], jnp.float32)
        c1_ref[...] = jnp.full((1, 128), ms_ref[2] + ms_ref[3], jnp.float32)


def _prep(xp, Wcat):
    NP, F = xp.shape
    grid = (NP // _BLK,)
    tab = pl.BlockSpec((_BLK, 128), lambda i: (i, 0))
    tshape = jax.ShapeDtypeStruct((NP, 128), jnp.float32)
    cspec = pl.BlockSpec((1, 128), lambda i: (0, 0))
    cshape = jax.ShapeDtypeStruct((1, 128), jnp.float32)
    return pl.pallas_call(
        _prep_body,
        grid=grid,
        in_specs=[
            pl.BlockSpec((_BLK, F), lambda i: (i, 0)),
            pl.BlockSpec((F, 512), lambda i: (0, 0)),
        ],
        out_specs=[tab, tab, tab, tab, cspec, cspec],
        out_shape=[tshape, tshape, tshape, tshape, cshape, cshape],
        scratch_shapes=[pltpu.SMEM((4,), jnp.float32)],
    )(xp, Wcat)


def _make_edge_fn(NP, EP, nh, mw):
    nw = _NCORE * _NSUB
    chunks_pw = EP // (nw * _CH)
    npairs = chunks_pw // 2
    assert chunks_pw % 2 == 0
    rows_ps = NP // _NSUB
    zrows = _CH
    assert rows_ps % zrows == 0
    zch = rows_ps // zrows
    mesh = plsc.VectorSubcoreMesh(core_axis_name="c", subcore_axis_name="s",
                                  num_cores=_NCORE, num_subcores=_NSUB)

    def body(src_ref, dst_ref, tabA_ref, tabD_ref, c_ref, accp_ref,
             acc, vsrc0, vsrc1, vdst0, vdst1, wdst0, wdst1,
             rowsA0, rowsA1, rowsD0, rowsD1, wbuf0, wbuf1, cbuf,
             semA0, semA1, semD0, semD1, semW0, semW1,
             semIS0, semIS1, semID0, semID1):
        cid = lax.axis_index("c")
        sid = lax.axis_index("s")
        vsrc = (vsrc0, vsrc1)
        vdst = (vdst0, vdst1)
        wdst = (wdst0, wdst1)
        rowsA = (rowsA0, rowsA1)
        rowsD = (rowsD0, rowsD1)
        wbuf = (wbuf0, wbuf1)
        semA = (semA0, semA1)
        semD = (semD0, semD1)
        semW = (semW0, semW1)
        semIS = (semIS0, semIS1)
        semID = (semID0, semID1)

        def zrow(i, _):
            for j in range(8):
                wbuf0[i, pl.ds(j * 16, 16)] = jnp.zeros((16,), jnp.float32)
                wbuf1[i, pl.ds(j * 16, 16)] = jnp.zeros((16,), jnp.float32)
            return 0

        lax.fori_loop(0, _CH, zrow, 0)
        for j in range(_CH // 16):
            wdst0[pl.ds(j * 16, 16)] = jnp.zeros((16,), jnp.int32)
            wdst1[pl.ds(j * 16, 16)] = jnp.zeros((16,), jnp.int32)

        def zacc(k, _):
            r = sid * rows_ps + k * zrows
            pltpu.sync_copy(wbuf0.at[pl.ds(0, zrows)], acc.at[pl.ds(r, zrows)])
            return 0

        lax.fori_loop(0, zch, zacc, 0)
        pltpu.sync_copy(c_ref, cbuf)
        plsc.subcore_barrier()
        cvec = cbuf[...]
        wid = cid * _NSUB + sid
        wbase = wid * chunks_pw

        pltpu.async_copy(wbuf0, acc.at[wdst0], semW0, add=True)
        pltpu.async_copy(wbuf1, acc.at[wdst1], semW1, add=True)
        pltpu.sync_copy(src_ref.at[pl.ds(wbase * _CH, _CH)], vsrc0)
        pltpu.sync_copy(dst_ref.at[pl.ds(wbase * _CH, _CH)], vdst0)
        pltpu.async_copy(src_ref.at[pl.ds((wbase + 1) * _CH, _CH)], vsrc1,
                         semIS1)
        pltpu.async_copy(dst_ref.at[pl.ds((wbase + 1) * _CH, _CH)], vdst1,
                         semID1)
        pltpu.async_copy(tabA_ref.at[vsrc0], rowsA0, semA0)
        pltpu.async_copy(tabD_ref.at[vdst0], rowsD0, semD0)

        def pair(p, _):
            for b in range(2):
                nb = 1 - b
                g = 2 * p + b
                gn = lax.rem(g + 1, chunks_pw)
                base = (wbase + gn) * _CH
                pltpu.make_async_copy(src_ref.at[pl.ds(base, _CH)],
                                      vsrc[nb], semIS[nb]).wait()
                pltpu.make_async_copy(dst_ref.at[pl.ds(base, _CH)],
                                      vdst[nb], semID[nb]).wait()
                pltpu.async_copy(tabA_ref.at[vsrc[nb]], rowsA[nb], semA[nb])
                pltpu.async_copy(tabD_ref.at[vdst[nb]], rowsD[nb], semD[nb])
                pltpu.make_async_copy(tabA_ref.at[vsrc[b]], rowsA[b],
                                      semA[b]).wait()
                pltpu.make_async_copy(tabD_ref.at[vdst[b]], rowsD[b],
                                      semD[b]).wait()
                pltpu.make_async_copy(wbuf[b], acc.at[wdst[b]],
                                      semW[b]).wait()
                for j in range(_CH // 16):
                    wdst[b][pl.ds(j * 16, 16)] = vdst[b][pl.ds(j * 16, 16)]
                g2 = lax.rem(g + 2, chunks_pw)
                base2 = (wbase + g2) * _CH
                pltpu.async_copy(src_ref.at[pl.ds(base2, _CH)], vsrc[b],
                                 semIS[b])
                pltpu.async_copy(dst_ref.at[pl.ds(base2, _CH)], vdst[b],
                                 semID[b])

                def erow(i, _):
                    t = (rowsA[b][i, pl.ds(64, 16)]
                         + rowsD[b][i, pl.ds(64, 16)])
                    alpha = jnp.maximum(t, 0.2 * t)
                    e16 = jnp.exp(alpha - cvec)
                    wbuf[b][i, pl.ds(64, 16)] = e16
                    for hh in range(nh):
                        ev = e16[hh]
                        for j2 in range(mw // 16):
                            off = hh * mw + j2 * 16
                            wbuf[b][i, pl.ds(off, 16)] = (
                                rowsA[b][i, pl.ds(off, 16)] * ev)
                    return 0

                lax.fori_loop(0, _CH, erow, 0, unroll=4)
                pltpu.async_copy(wbuf[b], acc.at[wdst[b]], semW[b], add=True)
            return 0

        lax.fori_loop(0, npairs, pair, 0)
        pltpu.make_async_copy(src_ref.at[pl.ds(wbase * _CH, _CH)], vsrc1,
                              semIS1).wait()
        pltpu.make_async_copy(dst_ref.at[pl.ds(wbase * _CH, _CH)], vdst1,
                              semID1).wait()
        pltpu.make_async_copy(tabA_ref.at[vsrc0], rowsA0, semA0).wait()
        pltpu.make_async_copy(tabD_ref.at[vdst0], rowsD0, semD0).wait()
        pltpu.make_async_copy(wbuf0, acc.at[wdst0], semW0).wait()
        pltpu.make_async_copy(wbuf1, acc.at[wdst1], semW1).wait()
        plsc.subcore_barrier()

        def wout(k, _):
            r = sid * rows_ps + k * zrows
            pltpu.sync_copy(acc.at[pl.ds(r, zrows)],
                            accp_ref.at[cid, pl.ds(r, zrows)])
            return 0

        lax.fori_loop(0, zch, wout, 0)

    return pl.kernel(
        body,
        out_type=jax.ShapeDtypeStruct((_NCORE, NP, 128), jnp.float32),
        mesh=mesh,
        scratch_types=[
            pltpu.VMEM_SHARED((NP, 128), jnp.float32),
            pltpu.VMEM((_CH,), jnp.int32),
            pltpu.VMEM((_CH,), jnp.int32),
            pltpu.VMEM((_CH,), jnp.int32),
            pltpu.VMEM((_CH,), jnp.int32),
            pltpu.VMEM((_CH,), jnp.int32),
            pltpu.VMEM((_CH,), jnp.int32),
            pltpu.VMEM((_CH, 128), jnp.float32),
            pltpu.VMEM((_CH, 128), jnp.float32),
            pltpu.VMEM((_CH, 128), jnp.float32),
            pltpu.VMEM((_CH, 128), jnp.float32),
            pltpu.VMEM((_CH, 128), jnp.float32),
            pltpu.VMEM((_CH, 128), jnp.float32),
            pltpu.VMEM((16,), jnp.float32),
            pltpu.SemaphoreType.DMA,
            pltpu.SemaphoreType.DMA,
            pltpu.SemaphoreType.DMA,
            pltpu.SemaphoreType.DMA,
            pltpu.SemaphoreType.DMA,
            pltpu.SemaphoreType.DMA,
            pltpu.SemaphoreType.DMA,
            pltpu.SemaphoreType.DMA,
            pltpu.SemaphoreType.DMA,
            pltpu.SemaphoreType.DMA,
        ],
    )


def _mid_body(p00_ref, p01_ref, p10_ref, p11_ref, rd_ref, b1_ref,
              wa_ref, wd_ref,
              out1_ref, tA2_ref, tD2_ref, c_ref, ms_ref):
    i = pl.program_id(0)
    g = pl.num_programs(0)
    q0 = p00_ref[...] + p01_ref[...]
    q1 = p10_ref[...] + p11_ref[...]
    den0 = jnp.dot(q0, rd_ref[...], preferred_element_type=jnp.float32)
    den1 = jnp.dot(q1, rd_ref[...], preferred_element_type=jnp.float32)
    o = jnp.concatenate(
        [q0[:, :64] / (den0 + 1e-16), q1[:, :64] / (den1 + 1e-16)], axis=1)
    o = o + b1_ref[...]
    out1_ref[...] = o
    gact = jnp.where(o > 0, o, jnp.exp(o) - 1.0)
    tA2 = jnp.dot(gact, wa_ref[...], preferred_element_type=jnp.float32)
    tD2 = jnp.dot(gact, wd_ref[...], preferred_element_type=jnp.float32)
    tA2_ref[...] = tA2
    tD2_ref[...] = tD2
    prev_s = jnp.where(i == 0, -jnp.inf, ms_ref[0])
    prev_d = jnp.where(i == 0, -jnp.inf, ms_ref[1])
    ms_ref[0] = jnp.maximum(prev_s, jnp.max(tA2[:, 64:65]))
    ms_ref[1] = jnp.maximum(prev_d, jnp.max(tD2[:, 64:65]))

    @pl.when(i == g - 1)
    def _():
        c_ref[...] = jnp.full((1, 128), ms_ref[0] + ms_ref[1], jnp.float32)


def _mid(p00, p01, p10, p11, Rden, b1r, WA2, WD2):
    NP = p00.shape[0]
    grid = (NP // _BLK,)
    blk = pl.BlockSpec((_BLK, 128), lambda i: (i, 0))
    shp = jax.ShapeDtypeStruct((NP, 128), jnp.float32)
    return pl.pallas_call(
        _mid_body,
        grid=grid,
        in_specs=[
            blk, blk, blk, blk,
            pl.BlockSpec((128, 64), lambda i: (0, 0)),
            pl.BlockSpec((1, 128), lambda i: (0, 0)),
            pl.BlockSpec((128, 128), lambda i: (0, 0)),
            pl.BlockSpec((128, 128), lambda i: (0, 0)),
        ],
        out_specs=[blk, blk, blk, pl.BlockSpec((1, 128), lambda i: (0, 0))],
        out_shape=[shp, shp, shp,
                   jax.ShapeDtypeStruct((1, 128), jnp.float32)],
        scratch_shapes=[pltpu.SMEM((2,), jnp.float32)],
    )(p00, p01, p10, p11, Rden, b1r, WA2, WD2)


def _fin_body(p0_ref, p1_ref, rd_ref, b2_ref, out_ref):
    q = p0_ref[...] + p1_ref[...]
    den = jnp.dot(q, rd_ref[...], preferred_element_type=jnp.float32)
    o2 = q[:, :64] / (den + 1e-16) + b2_ref[...]
    m = jnp.max(o2, axis=1, keepdims=True)
    ex = jnp.exp(o2 - m)
    lse = jnp.log(jnp.sum(ex, axis=1, keepdims=True))
    out_ref[...] = o2 - m - lse


def _fin(p0, p1, Rden2, b2r):
    NP = p0.shape[0]
    grid = (NP // _BLK,)
    blk = pl.BlockSpec((_BLK, 128), lambda i: (i, 0))
    return pl.pallas_call(
        _fin_body,
        grid=grid,
        in_specs=[
            blk, blk,
            pl.BlockSpec((128, 64), lambda i: (0, 0)),
            pl.BlockSpec((1, 64), lambda i: (0, 0)),
        ],
        out_specs=[pl.BlockSpec((_BLK, 64), lambda i: (i, 0))],
        out_shape=[jax.ShapeDtypeStruct((NP, 64), jnp.float32)],
    )(p0, p1, Rden2, b2r)


def kernel(x, edge_index, encoder_type, W1, att_src1, att_dst1, b1,
           W2, att_src2, att_dst2, b2):
    del encoder_type
    N, F = x.shape
    E = edge_index.shape[1]
    H, HD = att_src1.shape
    D1 = H * HD
    NC = W2.shape[1]

    npad = _NSUB * _CH
    NP = -(-N // npad) * npad
    epad = 2 * _NCORE * _NSUB * _CH
    EP = -(-E // epad) * epad

    xp = jnp.zeros((NP, F), jnp.float32).at[:N].set(x)
    fill = jnp.full((EP - E,), N, jnp.int32)
    srcp = jnp.concatenate([edge_index[0], fill])
    dstp = jnp.concatenate([edge_index[1], fill])

    eyeH = jnp.eye(H, 8, dtype=jnp.float32)
    Ablk_s = (att_src1[:, :, None] * eyeH[:, None, :]).reshape(D1, 8)
    Ablk_d = (att_dst1[:, :, None] * eyeH[:, None, :]).reshape(D1, 8)
    w1s = W1 @ Ablk_s
    w1d = W1 @ Ablk_d
    z56 = jnp.zeros((F, 56), jnp.float32)
    z60 = jnp.zeros((F, 60), jnp.float32)
    z64 = jnp.zeros((F, 64), jnp.float32)
    tA0 = jnp.concatenate([W1[:, 0:64], w1s[:, 0:4], z60], axis=1)
    tD0 = jnp.concatenate([z64, w1d[:, 0:4], z60], axis=1)
    tA1 = jnp.concatenate([W1[:, 64:128], w1s[:, 4:8], z60], axis=1)
    tD1 = jnp.concatenate([z64, w1d[:, 4:8], z60], axis=1)
    Wcat = jnp.concatenate([tA0, tD0, tA1, tD1], axis=1)

    w2s = W2 @ att_src2.T
    w2d = W2 @ att_dst2.T
    z63 = jnp.zeros((D1, 63), jnp.float32)
    WA2 = jnp.concatenate([W2, w2s, z63], axis=1)
    WD2 = jnp.concatenate([jnp.zeros((D1, 64), jnp.float32), w2d, z63],
                          axis=1)

    hd4 = jnp.eye(4, dtype=jnp.float32)
    Rden = jnp.zeros((128, 64), jnp.float32).at[64:68].set(
        jnp.repeat(hd4, HD, axis=1))
    Rden2 = jnp.zeros((128, 64), jnp.float32).at[64].set(1.0)
    b1r = b1.reshape(1, D1)
    b2r = b2.reshape(1, NC)

    tabA0, tabD0, tabA1, tabD1, c0m, c1m = _prep(xp, Wcat)
    edge4 = _make_edge_fn(NP, EP, 4, HD)
    p0 = edge4(srcp, dstp, tabA0, tabD0, c0m[0, :16])
    p1 = edge4(srcp, dstp, tabA1, tabD1, c1m[0, :16])
    out1f, tabA2, tabD2, c2m = _mid(p0[0], p0[1], p1[0], p1[1],
                                    Rden, b1r, WA2, WD2)
    p2 = _make_edge_fn(NP, EP, 1, NC)(srcp, dstp, tabA2, tabD2, c2m[0, :16])
    lsm = _fin(p2[0], p2[1], Rden2, b2r)[0]
    return (lsm[:N], out1f[:N])
